# baseline (speedup 1.0000x reference)
import functools

import jax
import jax.numpy as jnp
from jax import lax
from jax.experimental import pallas as pl
from jax.experimental.pallas import tpu as pltpu
from jax.experimental.pallas import tpu_sc as plsc

N, E, D = 10000, 320000, 128
NC, NS = 2, 16
EPT = E // NS
CH = 80
NCH = EPT // CH
SPLIT = 5040
NROWS = 5120
RPT = NROWS // NS
NPAD = 10240
HPT = NPAD // NS
BR = 80


@functools.lru_cache(maxsize=None)
def _sc_mesh():
    return plsc.VectorSubcoreMesh(
        core_axis_name="c", subcore_axis_name="s",
        num_cores=NC, num_subcores=NS)


def _degrees_body(ed, z1, dego, degi, idx_v, ones_v, ho, hi):
    c = lax.axis_index("c")
    s = lax.axis_index("s")
    for i in range(CH // 16):
        ones_v[pl.ds(16 * i, 16)] = jnp.ones((16,), jnp.float32)
    pltpu.sync_copy(ed.at[c, 0, s], idx_v.at[0])
    pltpu.sync_copy(ed.at[c, 1, s], idx_v.at[1])
    pltpu.sync_copy(z1, ho.at[pl.ds(HPT * s, HPT)])
    pltpu.sync_copy(z1, hi.at[pl.ds(HPT * s, HPT)])
    plsc.subcore_barrier()

    def body(j, carry):
        pltpu.sync_copy(ones_v, ho.at[idx_v.at[0, j]], add=True)
        pltpu.sync_copy(ones_v, hi.at[idx_v.at[1, j]], add=True)
        return carry

    lax.fori_loop(0, NCH, body, 0)
    plsc.subcore_barrier()
    pltpu.sync_copy(ho.at[pl.ds(HPT * s, HPT)], dego.at[c, 0, pl.ds(HPT * s, HPT)])
    pltpu.sync_copy(hi.at[pl.ds(HPT * s, HPT)], degi.at[c, 0, pl.ds(HPT * s, HPT)])


@functools.lru_cache(maxsize=None)
def _degrees():
    return pl.kernel(
        _degrees_body,
        out_type=(jax.ShapeDtypeStruct((NC, 1, NPAD), jnp.float32),
                  jax.ShapeDtypeStruct((NC, 1, NPAD), jnp.float32)),
        mesh=_sc_mesh(),
        scratch_types=[
            pltpu.VMEM((2, NCH, CH), jnp.int32),
            pltpu.VMEM((CH,), jnp.float32),
            pltpu.VMEM_SHARED((NPAD,), jnp.float32),
            pltpu.VMEM_SHARED((NPAD,), jnp.float32),
        ],
    )


def _spmm_body(ed, t0, t1, z2, agg, idx_v, buf0, buf1, acc,
               sem0, sem1):
    c = lax.axis_index("c")
    s = lax.axis_index("s")
    pltpu.sync_copy(ed.at[c, 0, s], idx_v.at[0])

    def run(tab):
        for p in (0, 1):
            pltpu.sync_copy(ed.at[c, 1, s], idx_v.at[1])

            def remap(j, carry, p=p):
                for k in range(CH // 16):
                    v = idx_v[1, j, pl.ds(16 * k, 16)]
                    l = v - (SPLIT * p)
                    m = (l >= 0) & (l < SPLIT)
                    t = SPLIT + 4 * s + (v & 3)
                    idx_v[1, j, pl.ds(16 * k, 16)] = jnp.where(m, l, t)
                return carry

            lax.fori_loop(0, NCH, remap, 0)
            pltpu.sync_copy(z2, acc.at[pl.ds(RPT * s, RPT)])
            plsc.subcore_barrier()
            pltpu.async_copy(tab.at[idx_v.at[0, 0]], buf0, sem0)

            def body(i, carry, tab=tab):
                g = 2 * i
                pltpu.make_async_copy(tab.at[idx_v.at[0, g]], buf0, sem0).wait()
                pltpu.async_copy(tab.at[idx_v.at[0, g + 1]], buf1, sem1)
                pltpu.sync_copy(buf0, acc.at[idx_v.at[1, g]], add=True)
                pltpu.make_async_copy(
                    tab.at[idx_v.at[0, g + 1]], buf1, sem1).wait()

                @pl.when(g + 2 < NCH)
                def _start_next():
                    pltpu.async_copy(tab.at[idx_v.at[0, g + 2]], buf0, sem0)

                pltpu.sync_copy(buf1, acc.at[idx_v.at[1, g + 1]], add=True)
                return carry

            lax.fori_loop(0, NCH // 2, body, 0)
            plsc.subcore_barrier()
            pltpu.sync_copy(acc.at[pl.ds(RPT * s, RPT)],
                            agg.at[p, c, pl.ds(RPT * s, RPT)])

    @pl.when(c == 0)
    def _rel0():
        run(t0)

    @pl.when(c == 1)
    def _rel1():
        run(t1)


@functools.lru_cache(maxsize=None)
def _spmm():
    return pl.kernel(
        _spmm_body,
        out_type=jax.ShapeDtypeStruct((2, NC, NROWS, D), jnp.float32),
        mesh=_sc_mesh(),
        scratch_types=[
            pltpu.VMEM((2, NCH, CH), jnp.int32),
            pltpu.VMEM((CH, D), jnp.float32),
            pltpu.VMEM((CH, D), jnp.float32),
            pltpu.VMEM_SHARED((NROWS, D), jnp.float32),
            pltpu.SemaphoreType.DMA,
            pltpu.SemaphoreType.DMA,
        ],
    )


def _prescale_body(x_ref, degt_ref, xs0_ref, xs1_ref, nrm_ref):
    d = degt_ref[...]
    nrm = lax.rsqrt(jnp.maximum(d, 1.0))
    nrm_ref[...] = nrm
    x = x_ref[...]
    xs0_ref[...] = x * nrm[:, 0:1]
    xs1_ref[...] = x * nrm[:, 2:3]


_prescale = pl.pallas_call(
    _prescale_body,
    grid=(N // BR,),
    in_specs=[
        pl.BlockSpec((BR, D), lambda i: (i, 0)),
        pl.BlockSpec((BR, 4), lambda i: (i, 0)),
    ],
    out_specs=[pl.BlockSpec((BR, D), lambda i: (i, 0))] * 2
    + [pl.BlockSpec((BR, 4), lambda i: (i, 0))],
    out_shape=[jax.ShapeDtypeStruct((N, D), jnp.float32)] * 2
    + [jax.ShapeDtypeStruct((N, 4), jnp.float32)],
)


def _gru_body(agg_ref, nrm_ref, w00_ref, w01_ref, wih_ref, b00_ref, b01_ref,
              bih_ref, bhh_ref, hs0_ref, hs1_ref):
    a = agg_ref[...]
    nrm = nrm_ref[...]
    y0 = a[0, 0] * nrm[:, 1:2]
    y1 = a[0, 1] * nrm[:, 3:4]
    b0 = b00_ref[...] + b01_ref[...]
    h = jnp.dot(y0, w00_ref[...], preferred_element_type=jnp.float32)
    h += jnp.dot(y1, w01_ref[...], preferred_element_type=jnp.float32)
    h = jnp.maximum(h + b0, 0.0)
    g = lax.dot_general(h, wih_ref[...], (((1,), (1,)), ((), ())),
                        preferred_element_type=jnp.float32)
    bih = bih_ref[...]
    bhh = bhh_ref[...]
    r = jax.nn.sigmoid(g[:, :D] + bih[0] + bhh[0])
    z = jax.nn.sigmoid(g[:, D:2 * D] + bih[1] + bhh[1])
    n = jnp.tanh(g[:, 2 * D:] + bih[2] + r * bhh[2])
    h2 = (1.0 - z) * n
    hs0_ref[...] = h2 * nrm[:, 0:1]
    hs1_ref[...] = h2 * nrm[:, 2:3]


_gru = pl.pallas_call(
    _gru_body,
    grid=(N // BR,),
    in_specs=[
        pl.BlockSpec((1, 2, BR, D), lambda i: (i // 63, 0, i - 63 * (i // 63), 0)),
        pl.BlockSpec((BR, 4), lambda i: (i, 0)),
        pl.BlockSpec((D, D), lambda i: (0, 0)),
        pl.BlockSpec((D, D), lambda i: (0, 0)),
        pl.BlockSpec((3 * D, D), lambda i: (0, 0)),
        pl.BlockSpec((1, D), lambda i: (0, 0)),
        pl.BlockSpec((1, D), lambda i: (0, 0)),
        pl.BlockSpec((3, D), lambda i: (0, 0)),
        pl.BlockSpec((3, D), lambda i: (0, 0)),
    ],
    out_specs=[pl.BlockSpec((BR, D), lambda i: (i, 0))] * 2,
    out_shape=[jax.ShapeDtypeStruct((N, D), jnp.float32)] * 2,
)


def _final_body(q_ref, nrm_ref, w20_ref, w21_ref, b20_ref, b21_ref, out_ref):
    q = q_ref[...]
    nrm = nrm_ref[...]
    y0 = q[0, 0] * nrm[:, 1:2]
    y1 = q[0, 1] * nrm[:, 3:4]
    out = jnp.dot(y0, w20_ref[...], preferred_element_type=jnp.float32)
    out += jnp.dot(y1, w21_ref[...], preferred_element_type=jnp.float32)
    out_ref[...] = out + b20_ref[...] + b21_ref[...]


_final = pl.pallas_call(
    _final_body,
    grid=(N // BR,),
    in_specs=[
        pl.BlockSpec((1, 2, BR, D), lambda i: (i // 63, 0, i - 63 * (i // 63), 0)),
        pl.BlockSpec((BR, 4), lambda i: (i, 0)),
        pl.BlockSpec((D, D), lambda i: (0, 0)),
        pl.BlockSpec((D, D), lambda i: (0, 0)),
        pl.BlockSpec((1, D), lambda i: (0, 0)),
        pl.BlockSpec((1, D), lambda i: (0, 0)),
    ],
    out_specs=pl.BlockSpec((BR, D), lambda i: (i, 0)),
    out_shape=jax.ShapeDtypeStruct((N, D), jnp.float32),
)


def kernel(x, edge_index_rel0, edge_index_rel1, hidden0, W0_0, b0_0, W0_1,
           b0_1, W2_0, b2_0, W2_1, b2_1, W_ih, W_hh, b_ih, b_hh):
    del hidden0, W_hh
    ed = jnp.stack([edge_index_rel0, edge_index_rel1])
    ed = ed.reshape(NC, 2, NS, NCH, CH)
    z1 = jnp.zeros((HPT,), jnp.float32)
    z2 = jnp.zeros((RPT, D), jnp.float32)

    dego, degi = _degrees()(ed, z1)
    degt = jnp.stack([dego[0, 0, :N], degi[0, 0, :N],
                      dego[1, 0, :N], degi[1, 0, :N]], axis=1)
    xs0, xs1, nrm = _prescale(x, degt)
    agg = _spmm()(ed, xs0, xs1, z2)
    hs0, hs1 = _gru(agg, nrm, W0_0, W0_1, W_ih,
                    b0_0.reshape(1, D), b0_1.reshape(1, D),
                    b_ih.reshape(3, D), b_hh.reshape(3, D))
    q = _spmm()(ed, hs0, hs1, z2)
    return _final(q, nrm, W2_0, W2_1,
                  b2_0.reshape(1, D), b2_1.reshape(1, D))


# speedup vs baseline: 1.0104x; 1.0104x over previous
import functools

import jax
import jax.numpy as jnp
from jax import lax
from jax.experimental import pallas as pl
from jax.experimental.pallas import tpu as pltpu
from jax.experimental.pallas import tpu_sc as plsc

N, E, D = 10000, 320000, 128
NC, NS = 2, 16
EPT = E // NS
CH = 80
NCH = EPT // CH
SPLIT = 5040
NROWS = 5120
RPT = NROWS // NS
NPAD = 10240
HPT = NPAD // NS
BR = 80


@functools.lru_cache(maxsize=None)
def _sc_mesh():
    return plsc.VectorSubcoreMesh(
        core_axis_name="c", subcore_axis_name="s",
        num_cores=NC, num_subcores=NS)


def _degrees_body(ed, z1, dego, degi, idx_v, ones_v, ho, hi):
    c = lax.axis_index("c")
    s = lax.axis_index("s")
    for i in range(CH // 16):
        ones_v[pl.ds(16 * i, 16)] = jnp.ones((16,), jnp.float32)
    pltpu.sync_copy(ed.at[c, 0, s], idx_v.at[0])
    pltpu.sync_copy(ed.at[c, 1, s], idx_v.at[1])
    pltpu.sync_copy(z1, ho.at[pl.ds(HPT * s, HPT)])
    pltpu.sync_copy(z1, hi.at[pl.ds(HPT * s, HPT)])
    plsc.subcore_barrier()

    def body(j, carry):
        pltpu.sync_copy(ones_v, ho.at[idx_v.at[0, j]], add=True)
        pltpu.sync_copy(ones_v, hi.at[idx_v.at[1, j]], add=True)
        return carry

    lax.fori_loop(0, NCH, body, 0)
    plsc.subcore_barrier()
    pltpu.sync_copy(ho.at[pl.ds(HPT * s, HPT)], dego.at[c, 0, pl.ds(HPT * s, HPT)])
    pltpu.sync_copy(hi.at[pl.ds(HPT * s, HPT)], degi.at[c, 0, pl.ds(HPT * s, HPT)])


@functools.lru_cache(maxsize=None)
def _degrees():
    return pl.kernel(
        _degrees_body,
        out_type=(jax.ShapeDtypeStruct((NC, 1, NPAD), jnp.float32),
                  jax.ShapeDtypeStruct((NC, 1, NPAD), jnp.float32)),
        mesh=_sc_mesh(),
        scratch_types=[
            pltpu.VMEM((2, NCH, CH), jnp.int32),
            pltpu.VMEM((CH,), jnp.float32),
            pltpu.VMEM_SHARED((NPAD,), jnp.float32),
            pltpu.VMEM_SHARED((NPAD,), jnp.float32),
        ],
    )


def _spmm_body(ed, t0, t1, z2, agg, idx_v, buf0, buf1, acc,
               sem0, sem1, sem2, sem3):
    c = lax.axis_index("c")
    s = lax.axis_index("s")
    pltpu.sync_copy(ed.at[c, 0, s], idx_v.at[0])

    def run(tab):
        for p in (0, 1):
            pltpu.sync_copy(ed.at[c, 1, s], idx_v.at[1])

            def remap(j, carry, p=p):
                for k in range(CH // 16):
                    v = idx_v[1, j, pl.ds(16 * k, 16)]
                    l = v - (SPLIT * p)
                    m = (l >= 0) & (l < SPLIT)
                    t = SPLIT + 4 * s + (v & 3)
                    idx_v[1, j, pl.ds(16 * k, 16)] = jnp.where(m, l, t)
                return carry

            lax.fori_loop(0, NCH, remap, 0)
            pltpu.sync_copy(z2, acc.at[pl.ds(RPT * s, RPT)])
            plsc.subcore_barrier()
            pltpu.async_copy(tab.at[idx_v.at[0, 0]], buf0, sem0)
            pltpu.async_copy(tab.at[idx_v.at[0, 1]], buf1, sem1)

            def body(i, carry, tab=tab):
                g = 2 * i
                pltpu.make_async_copy(tab.at[idx_v.at[0, g]], buf0, sem0).wait()
                pltpu.async_copy(buf0, acc.at[idx_v.at[1, g]], sem2, add=True)
                pltpu.make_async_copy(
                    tab.at[idx_v.at[0, g + 1]], buf1, sem1).wait()
                pltpu.async_copy(buf1, acc.at[idx_v.at[1, g + 1]], sem3,
                                 add=True)
                pltpu.make_async_copy(
                    buf0, acc.at[idx_v.at[1, g]], sem2).wait()

                @pl.when(g + 2 < NCH)
                def _start_next0():
                    pltpu.async_copy(tab.at[idx_v.at[0, g + 2]], buf0, sem0)

                pltpu.make_async_copy(
                    buf1, acc.at[idx_v.at[1, g + 1]], sem3).wait()

                @pl.when(g + 3 < NCH)
                def _start_next1():
                    pltpu.async_copy(tab.at[idx_v.at[0, g + 3]], buf1, sem1)

                return carry

            lax.fori_loop(0, NCH // 2, body, 0)
            plsc.subcore_barrier()
            pltpu.sync_copy(acc.at[pl.ds(RPT * s, RPT)],
                            agg.at[p, c, pl.ds(RPT * s, RPT)])

    @pl.when(c == 0)
    def _rel0():
        run(t0)

    @pl.when(c == 1)
    def _rel1():
        run(t1)


@functools.lru_cache(maxsize=None)
def _spmm():
    return pl.kernel(
        _spmm_body,
        out_type=jax.ShapeDtypeStruct((2, NC, NROWS, D), jnp.float32),
        mesh=_sc_mesh(),
        scratch_types=[
            pltpu.VMEM((2, NCH, CH), jnp.int32),
            pltpu.VMEM((CH, D), jnp.float32),
            pltpu.VMEM((CH, D), jnp.float32),
            pltpu.VMEM_SHARED((NROWS, D), jnp.float32),
            pltpu.SemaphoreType.DMA,
            pltpu.SemaphoreType.DMA,
            pltpu.SemaphoreType.DMA,
            pltpu.SemaphoreType.DMA,
        ],
    )


def _prescale_body(x_ref, degt_ref, xs0_ref, xs1_ref, nrm_ref):
    d = degt_ref[...]
    nrm = lax.rsqrt(jnp.maximum(d, 1.0))
    nrm_ref[...] = nrm
    x = x_ref[...]
    xs0_ref[...] = x * nrm[:, 0:1]
    xs1_ref[...] = x * nrm[:, 2:3]


_prescale = pl.pallas_call(
    _prescale_body,
    grid=(N // BR,),
    in_specs=[
        pl.BlockSpec((BR, D), lambda i: (i, 0)),
        pl.BlockSpec((BR, 4), lambda i: (i, 0)),
    ],
    out_specs=[pl.BlockSpec((BR, D), lambda i: (i, 0))] * 2
    + [pl.BlockSpec((BR, 4), lambda i: (i, 0))],
    out_shape=[jax.ShapeDtypeStruct((N, D), jnp.float32)] * 2
    + [jax.ShapeDtypeStruct((N, 4), jnp.float32)],
)


def _gru_body(agg_ref, nrm_ref, w00_ref, w01_ref, wih_ref, b00_ref, b01_ref,
              bih_ref, bhh_ref, hs0_ref, hs1_ref):
    a = agg_ref[...]
    nrm = nrm_ref[...]
    y0 = a[0, 0] * nrm[:, 1:2]
    y1 = a[0, 1] * nrm[:, 3:4]
    b0 = b00_ref[...] + b01_ref[...]
    h = jnp.dot(y0, w00_ref[...], preferred_element_type=jnp.float32)
    h += jnp.dot(y1, w01_ref[...], preferred_element_type=jnp.float32)
    h = jnp.maximum(h + b0, 0.0)
    g = lax.dot_general(h, wih_ref[...], (((1,), (1,)), ((), ())),
                        preferred_element_type=jnp.float32)
    bih = bih_ref[...]
    bhh = bhh_ref[...]
    r = jax.nn.sigmoid(g[:, :D] + bih[0] + bhh[0])
    z = jax.nn.sigmoid(g[:, D:2 * D] + bih[1] + bhh[1])
    n = jnp.tanh(g[:, 2 * D:] + bih[2] + r * bhh[2])
    h2 = (1.0 - z) * n
    hs0_ref[...] = h2 * nrm[:, 0:1]
    hs1_ref[...] = h2 * nrm[:, 2:3]


_gru = pl.pallas_call(
    _gru_body,
    grid=(N // BR,),
    in_specs=[
        pl.BlockSpec((1, 2, BR, D), lambda i: (i // 63, 0, i - 63 * (i // 63), 0)),
        pl.BlockSpec((BR, 4), lambda i: (i, 0)),
        pl.BlockSpec((D, D), lambda i: (0, 0)),
        pl.BlockSpec((D, D), lambda i: (0, 0)),
        pl.BlockSpec((3 * D, D), lambda i: (0, 0)),
        pl.BlockSpec((1, D), lambda i: (0, 0)),
        pl.BlockSpec((1, D), lambda i: (0, 0)),
        pl.BlockSpec((3, D), lambda i: (0, 0)),
        pl.BlockSpec((3, D), lambda i: (0, 0)),
    ],
    out_specs=[pl.BlockSpec((BR, D), lambda i: (i, 0))] * 2,
    out_shape=[jax.ShapeDtypeStruct((N, D), jnp.float32)] * 2,
)


def _final_body(q_ref, nrm_ref, w20_ref, w21_ref, b20_ref, b21_ref, out_ref):
    q = q_ref[...]
    nrm = nrm_ref[...]
    y0 = q[0, 0] * nrm[:, 1:2]
    y1 = q[0, 1] * nrm[:, 3:4]
    out = jnp.dot(y0, w20_ref[...], preferred_element_type=jnp.float32)
    out += jnp.dot(y1, w21_ref[...], preferred_element_type=jnp.float32)
    out_ref[...] = out + b20_ref[...] + b21_ref[...]


_final = pl.pallas_call(
    _final_body,
    grid=(N // BR,),
    in_specs=[
        pl.BlockSpec((1, 2, BR, D), lambda i: (i // 63, 0, i - 63 * (i // 63), 0)),
        pl.BlockSpec((BR, 4), lambda i: (i, 0)),
        pl.BlockSpec((D, D), lambda i: (0, 0)),
        pl.BlockSpec((D, D), lambda i: (0, 0)),
        pl.BlockSpec((1, D), lambda i: (0, 0)),
        pl.BlockSpec((1, D), lambda i: (0, 0)),
    ],
    out_specs=pl.BlockSpec((BR, D), lambda i: (i, 0)),
    out_shape=jax.ShapeDtypeStruct((N, D), jnp.float32),
)


def kernel(x, edge_index_rel0, edge_index_rel1, hidden0, W0_0, b0_0, W0_1,
           b0_1, W2_0, b2_0, W2_1, b2_1, W_ih, W_hh, b_ih, b_hh):
    del hidden0, W_hh
    ed = jnp.stack([edge_index_rel0, edge_index_rel1])
    ed = ed.reshape(NC, 2, NS, NCH, CH)
    z1 = jnp.zeros((HPT,), jnp.float32)
    z2 = jnp.zeros((RPT, D), jnp.float32)

    dego, degi = _degrees()(ed, z1)
    degt = jnp.stack([dego[0, 0, :N], degi[0, 0, :N],
                      dego[1, 0, :N], degi[1, 0, :N]], axis=1)
    xs0, xs1, nrm = _prescale(x, degt)
    agg = _spmm()(ed, xs0, xs1, z2)
    hs0, hs1 = _gru(agg, nrm, W0_0, W0_1, W_ih,
                    b0_0.reshape(1, D), b0_1.reshape(1, D),
                    b_ih.reshape(3, D), b_hh.reshape(3, D))
    q = _spmm()(ed, hs0, hs1, z2)
    return _final(q, nrm, W2_0, W2_1,
                  b2_0.reshape(1, D), b2_1.reshape(1, D))


# speedup vs baseline: 1.1433x; 1.1315x over previous
import functools

import jax
import jax.numpy as jnp
from jax import lax
from jax.experimental import pallas as pl
from jax.experimental.pallas import tpu as pltpu
from jax.experimental.pallas import tpu_sc as plsc

N, E, D = 10000, 320000, 128
NC, NS = 2, 16
EPT = E // NS
CH = 80
NCH = EPT // CH
SPLIT = 5200
NROWS = 5248
RPT = NROWS // NS
NPAD = 10240
HPT = NPAD // NS
BR = 400


@functools.lru_cache(maxsize=None)
def _sc_mesh():
    return plsc.VectorSubcoreMesh(
        core_axis_name="c", subcore_axis_name="s",
        num_cores=NC, num_subcores=NS)


def _degrees_body(ed, z1, dego, degi, idx_v, ones_v, ho, hi):
    c = lax.axis_index("c")
    s = lax.axis_index("s")
    for i in range(CH // 16):
        ones_v[pl.ds(16 * i, 16)] = jnp.ones((16,), jnp.float32)
    pltpu.sync_copy(ed.at[c, 0, s], idx_v.at[0])
    pltpu.sync_copy(ed.at[c, 1, s], idx_v.at[1])
    pltpu.sync_copy(z1, ho.at[pl.ds(HPT * s, HPT)])
    pltpu.sync_copy(z1, hi.at[pl.ds(HPT * s, HPT)])
    plsc.subcore_barrier()

    def body(j, carry):
        pltpu.sync_copy(ones_v, ho.at[idx_v.at[0, j]], add=True)
        pltpu.sync_copy(ones_v, hi.at[idx_v.at[1, j]], add=True)
        return carry

    lax.fori_loop(0, NCH, body, 0)
    plsc.subcore_barrier()
    pltpu.sync_copy(ho.at[pl.ds(HPT * s, HPT)], dego.at[c, 0, pl.ds(HPT * s, HPT)])
    pltpu.sync_copy(hi.at[pl.ds(HPT * s, HPT)], degi.at[c, 0, pl.ds(HPT * s, HPT)])


@functools.lru_cache(maxsize=None)
def _degrees():
    return pl.kernel(
        _degrees_body,
        out_type=(jax.ShapeDtypeStruct((NC, 1, NPAD), jnp.float32),
                  jax.ShapeDtypeStruct((NC, 1, NPAD), jnp.float32)),
        mesh=_sc_mesh(),
        scratch_types=[
            pltpu.VMEM((2, NCH, CH), jnp.int32),
            pltpu.VMEM((CH,), jnp.float32),
            pltpu.VMEM_SHARED((NPAD,), jnp.float32),
            pltpu.VMEM_SHARED((NPAD,), jnp.float32),
        ],
    )


def _spmm_body(ed, t0, t1, z2, agg, idx_v, buf0, buf1, acc,
               sem0, sem1, sem2, sem3):
    c = lax.axis_index("c")
    s = lax.axis_index("s")
    pltpu.sync_copy(ed.at[c, 0, s], idx_v.at[0])

    def run(tab):
        for p in (0, 1):
            pltpu.sync_copy(ed.at[c, 1, s], idx_v.at[1])

            def remap(j, carry, p=p):
                for k in range(CH // 16):
                    v = idx_v[1, j, pl.ds(16 * k, 16)]
                    l = v - (SPLIT * p)
                    m = (l >= 0) & (l < SPLIT)
                    t = SPLIT + (v & 31)
                    idx_v[1, j, pl.ds(16 * k, 16)] = jnp.where(m, l, t)
                return carry

            lax.fori_loop(0, NCH, remap, 0)
            pltpu.sync_copy(z2, acc.at[pl.ds(RPT * s, RPT)])
            plsc.subcore_barrier()
            pltpu.async_copy(tab.at[idx_v.at[0, 0]], buf0, sem0)
            pltpu.async_copy(tab.at[idx_v.at[0, 1]], buf1, sem1)

            def body(i, carry, tab=tab):
                g = 2 * i
                pltpu.make_async_copy(tab.at[idx_v.at[0, g]], buf0, sem0).wait()
                pltpu.async_copy(buf0, acc.at[idx_v.at[1, g]], sem2, add=True)
                pltpu.make_async_copy(
                    tab.at[idx_v.at[0, g + 1]], buf1, sem1).wait()
                pltpu.async_copy(buf1, acc.at[idx_v.at[1, g + 1]], sem3,
                                 add=True)
                pltpu.make_async_copy(
                    buf0, acc.at[idx_v.at[1, g]], sem2).wait()

                @pl.when(g + 2 < NCH)
                def _start_next0():
                    pltpu.async_copy(tab.at[idx_v.at[0, g + 2]], buf0, sem0)

                pltpu.make_async_copy(
                    buf1, acc.at[idx_v.at[1, g + 1]], sem3).wait()

                @pl.when(g + 3 < NCH)
                def _start_next1():
                    pltpu.async_copy(tab.at[idx_v.at[0, g + 3]], buf1, sem1)

                return carry

            lax.fori_loop(0, NCH // 2, body, 0)
            plsc.subcore_barrier()
            pltpu.sync_copy(acc.at[pl.ds(RPT * s, RPT)],
                            agg.at[p, c, pl.ds(RPT * s, RPT)])

    @pl.when(c == 0)
    def _rel0():
        run(t0)

    @pl.when(c == 1)
    def _rel1():
        run(t1)


@functools.lru_cache(maxsize=None)
def _spmm():
    return pl.kernel(
        _spmm_body,
        out_type=jax.ShapeDtypeStruct((2, NC, NROWS, D), jnp.float32),
        mesh=_sc_mesh(),
        scratch_types=[
            pltpu.VMEM((2, NCH, CH), jnp.int32),
            pltpu.VMEM((CH, D), jnp.float32),
            pltpu.VMEM((CH, D), jnp.float32),
            pltpu.VMEM_SHARED((NROWS, D), jnp.float32),
            pltpu.SemaphoreType.DMA,
            pltpu.SemaphoreType.DMA,
            pltpu.SemaphoreType.DMA,
            pltpu.SemaphoreType.DMA,
        ],
    )


def _prescale_body(x_ref, degt_ref, xs0_ref, xs1_ref, nrm_ref):
    d = degt_ref[...]
    nrm = lax.rsqrt(jnp.maximum(d, 1.0))
    nrm_ref[...] = nrm
    x = x_ref[...]
    xs0_ref[...] = x * nrm[:, 0:1]
    xs1_ref[...] = x * nrm[:, 2:3]


_prescale = pl.pallas_call(
    _prescale_body,
    grid=(N // BR,),
    in_specs=[
        pl.BlockSpec((BR, D), lambda i: (i, 0)),
        pl.BlockSpec((BR, 4), lambda i: (i, 0)),
    ],
    out_specs=[pl.BlockSpec((BR, D), lambda i: (i, 0))] * 2
    + [pl.BlockSpec((BR, 4), lambda i: (i, 0))],
    out_shape=[jax.ShapeDtypeStruct((N, D), jnp.float32)] * 2
    + [jax.ShapeDtypeStruct((N, 4), jnp.float32)],
)


def _gru_body(agg_ref, nrm_ref, w00_ref, w01_ref, wih_ref, b00_ref, b01_ref,
              bih_ref, bhh_ref, hs0_ref, hs1_ref):
    a = agg_ref[...]
    nrm = nrm_ref[...]
    y0 = a[0, 0] * nrm[:, 1:2]
    y1 = a[0, 1] * nrm[:, 3:4]
    b0 = b00_ref[...] + b01_ref[...]
    h = jnp.dot(y0, w00_ref[...], preferred_element_type=jnp.float32)
    h += jnp.dot(y1, w01_ref[...], preferred_element_type=jnp.float32)
    h = jnp.maximum(h + b0, 0.0)
    g = lax.dot_general(h, wih_ref[...], (((1,), (1,)), ((), ())),
                        preferred_element_type=jnp.float32)
    bih = bih_ref[...]
    bhh = bhh_ref[...]
    r = jax.nn.sigmoid(g[:, :D] + bih[0] + bhh[0])
    z = jax.nn.sigmoid(g[:, D:2 * D] + bih[1] + bhh[1])
    n = jnp.tanh(g[:, 2 * D:] + bih[2] + r * bhh[2])
    h2 = (1.0 - z) * n
    hs0_ref[...] = h2 * nrm[:, 0:1]
    hs1_ref[...] = h2 * nrm[:, 2:3]


_gru = pl.pallas_call(
    _gru_body,
    grid=(N // BR,),
    in_specs=[
        pl.BlockSpec((1, 2, BR, D), lambda i: (i // 13, 0, i - 13 * (i // 13), 0)),
        pl.BlockSpec((BR, 4), lambda i: (i, 0)),
        pl.BlockSpec((D, D), lambda i: (0, 0)),
        pl.BlockSpec((D, D), lambda i: (0, 0)),
        pl.BlockSpec((3 * D, D), lambda i: (0, 0)),
        pl.BlockSpec((1, D), lambda i: (0, 0)),
        pl.BlockSpec((1, D), lambda i: (0, 0)),
        pl.BlockSpec((3, D), lambda i: (0, 0)),
        pl.BlockSpec((3, D), lambda i: (0, 0)),
    ],
    out_specs=[pl.BlockSpec((BR, D), lambda i: (i, 0))] * 2,
    out_shape=[jax.ShapeDtypeStruct((N, D), jnp.float32)] * 2,
)


def _final_body(q_ref, nrm_ref, w20_ref, w21_ref, b20_ref, b21_ref, out_ref):
    q = q_ref[...]
    nrm = nrm_ref[...]
    y0 = q[0, 0] * nrm[:, 1:2]
    y1 = q[0, 1] * nrm[:, 3:4]
    out = jnp.dot(y0, w20_ref[...], preferred_element_type=jnp.float32)
    out += jnp.dot(y1, w21_ref[...], preferred_element_type=jnp.float32)
    out_ref[...] = out + b20_ref[...] + b21_ref[...]


_final = pl.pallas_call(
    _final_body,
    grid=(N // BR,),
    in_specs=[
        pl.BlockSpec((1, 2, BR, D), lambda i: (i // 13, 0, i - 13 * (i // 13), 0)),
        pl.BlockSpec((BR, 4), lambda i: (i, 0)),
        pl.BlockSpec((D, D), lambda i: (0, 0)),
        pl.BlockSpec((D, D), lambda i: (0, 0)),
        pl.BlockSpec((1, D), lambda i: (0, 0)),
        pl.BlockSpec((1, D), lambda i: (0, 0)),
    ],
    out_specs=pl.BlockSpec((BR, D), lambda i: (i, 0)),
    out_shape=jax.ShapeDtypeStruct((N, D), jnp.float32),
)


def kernel(x, edge_index_rel0, edge_index_rel1, hidden0, W0_0, b0_0, W0_1,
           b0_1, W2_0, b2_0, W2_1, b2_1, W_ih, W_hh, b_ih, b_hh):
    del hidden0, W_hh
    ed = jnp.stack([edge_index_rel0, edge_index_rel1])
    ed = ed.reshape(NC, 2, NS, NCH, CH)
    z1 = jnp.zeros((HPT,), jnp.float32)
    z2 = jnp.zeros((RPT, D), jnp.float32)

    dego, degi = _degrees()(ed, z1)
    degt = jnp.stack([dego[0, 0, :N], degi[0, 0, :N],
                      dego[1, 0, :N], degi[1, 0, :N]], axis=1)
    xs0, xs1, nrm = _prescale(x, degt)
    agg = _spmm()(ed, xs0, xs1, z2)
    hs0, hs1 = _gru(agg, nrm, W0_0, W0_1, W_ih,
                    b0_0.reshape(1, D), b0_1.reshape(1, D),
                    b_ih.reshape(3, D), b_hh.reshape(3, D))
    q = _spmm()(ed, hs0, hs1, z2)
    return _final(q, nrm, W2_0, W2_1,
                  b2_0.reshape(1, D), b2_1.reshape(1, D))
